# baseline (speedup 1.0000x reference)
import functools

import jax
import jax.numpy as jnp
from jax import lax
from jax.experimental import pallas as pl
from jax.experimental.pallas import tpu as pltpu
from jax.experimental.pallas import tpu_sc as plsc

_N_CLS = 1000
_N_CTX = 16
_DIM = 512
_SEQ = 77
_SUF = _SEQ - 1 - _N_CTX
_NCOPY = 1 + _SUF
_CH = 40
_NCHUNK = _N_CLS // _CH
_TOTAL = _NCOPY * _NCHUNK

_N_A = 54
_N_B = 41
_TAIL = _TOTAL - 16 * _N_A - 16 * _N_B
_NBUF = 5
_NBUF_B = 5


@functools.cache
def _build_sc_kernel():
    info = plsc.get_sparse_core_info()
    nc, ns = info.num_cores, info.num_subcores
    mesh = plsc.VectorSubcoreMesh(core_axis_name="c", subcore_axis_name="s")

    @functools.partial(
        pl.kernel,
        out_type=jax.ShapeDtypeStruct((_SEQ, _N_CLS, _DIM), jnp.float32),
        mesh=mesh,
        scratch_types=(
            [pltpu.VMEM((_CH, _DIM), jnp.float32) for _ in range(_NBUF + 1)]
            + [
                pltpu.SemaphoreType.DMA((_NBUF,)),
                pltpu.SemaphoreType.DMA((_NBUF,)),
                pltpu.SemaphoreType.DMA,
            ]
        ),
    )
    def prompts_kernel(ctx_hbm, pre_hbm, suf_hbm, out_hbm,
                       *scratch):
        bufs = scratch[:_NBUF]
        rep = scratch[_NBUF]
        lsem, ssem, csem = scratch[_NBUF + 1:]
        wid = lax.axis_index("s") * nc + lax.axis_index("c")

        def chunk_coords(g):
            plane = lax.div(g, _NCHUNK)
            off = lax.rem(g, _NCHUNK) * _CH
            dst = jnp.where(plane == 0, 0, plane + _N_CTX)
            return plane, off, dst

        def load(b, g):
            plane, off, _ = chunk_coords(g)

            @pl.when(plane == 0)
            def _():
                pltpu.make_async_copy(
                    pre_hbm.at[pl.ds(off, _CH)], bufs[b], lsem.at[b]).start()

            @pl.when(plane != 0)
            def _():
                pltpu.make_async_copy(
                    suf_hbm.at[plane - 1, pl.ds(off, _CH)], bufs[b],
                    lsem.at[b]).start()

        def wait_load(b):
            pltpu.make_async_copy(
                pre_hbm.at[pl.ds(0, _CH)], bufs[b], lsem.at[b]).wait()

        def store(b, g):
            _, off, dst = chunk_coords(g)
            pltpu.make_async_copy(
                bufs[b], out_hbm.at[dst, pl.ds(off, _CH)], ssem.at[b]).start()

        def wait_store(b):
            pltpu.make_async_copy(
                bufs[b], out_hbm.at[0, pl.ds(0, _CH)], ssem.at[b]).wait()

        def run_pipeline(g0, n, nbuf):
            assert n >= nbuf + 1
            for j in range(nbuf - 1):
                load(j, g0 + j)
            for i in range(n):
                b = i % nbuf
                wait_load(b)
                store(b, g0 + i)
                if i + nbuf - 1 < n:
                    nb = (i + nbuf - 1) % nbuf
                    if i >= 1:
                        wait_store(nb)
                    load(nb, g0 + i + nbuf - 1)
            for b in range(nbuf):
                wait_store(b)

        @pl.when(wid < _TAIL)
        def _():
            run_pipeline(wid * (_N_A + 1), _N_A + 1, _NBUF)

        @pl.when((wid >= _TAIL) & (wid < 16))
        def _():
            run_pipeline(_TAIL + wid * _N_A, _N_A, _NBUF)

        @pl.when(wid >= 16)
        def _():
            r = wid - 16
            def fill(k, carry):
                pltpu.make_async_copy(ctx_hbm.at[r], rep.at[k], csem).start()
                return carry

            lax.fori_loop(0, _CH, fill, 0)

            def fill_drain(k, carry):
                pltpu.make_async_copy(ctx_hbm.at[0], rep.at[0], csem).wait()
                return carry

            lax.fori_loop(0, _CH, fill_drain, 0)
            for j in range(_NCHUNK):
                pltpu.make_async_copy(
                    rep, out_hbm.at[r + 1, pl.ds(j * _CH, _CH)], csem).start()
            run_pipeline(16 * _N_A + _TAIL + (wid - 16) * _N_B, _N_B, _NBUF_B)
            for j in range(_NCHUNK):
                pltpu.make_async_copy(
                    rep, out_hbm.at[1, pl.ds(0, _CH)], csem).wait()

    return prompts_kernel


def kernel(ctx, token_prefix, token_suffix):
    out_t = _build_sc_kernel()(
        ctx,
        token_prefix.reshape(_N_CLS, _DIM),
        token_suffix.transpose(1, 0, 2),
    )
    return out_t.transpose(1, 0, 2)


# speedup vs baseline: 1.0484x; 1.0484x over previous
import functools

import jax
import jax.numpy as jnp
from jax import lax
from jax.experimental import pallas as pl
from jax.experimental.pallas import tpu as pltpu
from jax.experimental.pallas import tpu_sc as plsc

_N_CLS = 1000
_N_CTX = 16
_DIM = 512
_SEQ = 77
_SUF = _SEQ - 1 - _N_CTX
_NCOPY = 1 + _SUF
_CH = 40
_NCHUNK = _N_CLS // _CH
_TOTAL = _NCOPY * _NCHUNK

_N_A = 54
_N_B = 41
_TAIL = _TOTAL - 16 * _N_A - 16 * _N_B
_NBUF = 5
_NBUF_B = 5


@functools.cache
def _build_sc_kernel():
    info = plsc.get_sparse_core_info()
    nc, ns = info.num_cores, info.num_subcores
    mesh = plsc.VectorSubcoreMesh(core_axis_name="c", subcore_axis_name="s")

    @functools.partial(
        pl.kernel,
        out_type=jax.ShapeDtypeStruct((_SEQ, _N_CLS, _DIM), jnp.float32),
        mesh=mesh,
        scratch_types=(
            [pltpu.VMEM((_CH, _DIM), jnp.float32) for _ in range(_NBUF + 1)]
            + [
                pltpu.SemaphoreType.DMA((_NBUF,)),
                pltpu.SemaphoreType.DMA((_NBUF,)),
                pltpu.SemaphoreType.DMA,
            ]
        ),
    )
    def prompts_kernel(ctx_hbm, pre_hbm, suf_hbm, out_hbm,
                       *scratch):
        bufs = scratch[:_NBUF]
        rep = scratch[_NBUF]
        lsem, ssem, csem = scratch[_NBUF + 1:]
        wid = lax.axis_index("s") * nc + lax.axis_index("c")

        def chunk_coords(g):
            plane = lax.div(g, _NCHUNK)
            off = lax.rem(g, _NCHUNK) * _CH
            dst = jnp.where(plane == 0, 0, plane + _N_CTX)
            return plane, off, dst

        def load(b, g):
            plane, off, _ = chunk_coords(g)

            @pl.when(plane == 0)
            def _():
                pltpu.make_async_copy(
                    pre_hbm.at[pl.ds(off, _CH)], bufs[b], lsem.at[b]).start()

            @pl.when(plane != 0)
            def _():
                pltpu.make_async_copy(
                    suf_hbm.at[plane - 1, pl.ds(off, _CH)], bufs[b],
                    lsem.at[b]).start()

        def wait_load(b):
            pltpu.make_async_copy(
                pre_hbm.at[pl.ds(0, _CH)], bufs[b], lsem.at[b]).wait()

        def store(b, g):
            _, off, dst = chunk_coords(g)
            pltpu.make_async_copy(
                bufs[b], out_hbm.at[dst, pl.ds(off, _CH)], ssem.at[b]).start()

        def wait_store(b):
            pltpu.make_async_copy(
                bufs[b], out_hbm.at[0, pl.ds(0, _CH)], ssem.at[b]).wait()

        n = jnp.where(wid < _TAIL, _N_A + 1,
                      jnp.where(wid < 16, _N_A, _N_B))
        g0 = jnp.where(wid < _TAIL, wid * (_N_A + 1),
                       jnp.where(wid < 16, _TAIL + wid * _N_A,
                                 16 * _N_A + _TAIL + (wid - 16) * _N_B))
        n_max = _N_A + 1

        def run_pipeline(g0, n):
            for j in range(_NBUF - 1):
                load(j, g0 + j)
            for i in range(n_max):
                b = i % _NBUF

                @pl.when(i < n)
                def _(i=i, b=b):
                    wait_load(b)
                    store(b, g0 + i)

                j = i + _NBUF - 1
                if j < n_max:
                    nb = j % _NBUF

                    @pl.when(j < n)
                    def _(i=i, j=j, nb=nb):
                        if i >= 1:
                            wait_store(nb)
                        load(nb, g0 + j)
            for b in range(_NBUF):
                wait_store(b)

        @pl.when(wid >= 16)
        def _():
            r = wid - 16
            def fill(k, carry):
                pltpu.make_async_copy(ctx_hbm.at[r], rep.at[k], csem).start()
                return carry

            lax.fori_loop(0, _CH, fill, 0)

            def fill_drain(k, carry):
                pltpu.make_async_copy(ctx_hbm.at[0], rep.at[0], csem).wait()
                return carry

            lax.fori_loop(0, _CH, fill_drain, 0)
            for j in range(_NCHUNK):
                pltpu.make_async_copy(
                    rep, out_hbm.at[r + 1, pl.ds(j * _CH, _CH)], csem).start()

        run_pipeline(g0, n)

        @pl.when(wid >= 16)
        def _():
            for j in range(_NCHUNK):
                pltpu.make_async_copy(
                    rep, out_hbm.at[1, pl.ds(0, _CH)], csem).wait()

    return prompts_kernel


def kernel(ctx, token_prefix, token_suffix):
    out_t = _build_sc_kernel()(
        ctx,
        token_prefix.reshape(_N_CLS, _DIM),
        token_suffix.transpose(1, 0, 2),
    )
    return out_t.transpose(1, 0, 2)
